# baseline (speedup 1.0000x reference)
import jax
import jax.numpy as jnp
from jax.experimental import pallas as pl
from jax.experimental.pallas import tpu as pltpu

_C = 256
_NBUF = 4


def _add_pe_kernel(x_hbm, pe_hbm, o_hbm, xbuf, pebuf, insem, outsem):
    B, S, E = x_hbm.shape
    n = S // _C

    def in_copies(k):
        slot = k % _NBUF
        return [
            pltpu.make_async_copy(
                x_hbm.at[:, pl.ds(k * _C, _C), :], xbuf.at[slot],
                insem.at[slot]),
            pltpu.make_async_copy(
                pe_hbm.at[pl.ds(k * _C, _C), :], pebuf.at[slot],
                insem.at[slot]),
        ]

    def out_copy(k):
        slot = k % _NBUF
        return pltpu.make_async_copy(
            xbuf.at[slot], o_hbm.at[:, pl.ds(k * _C, _C), :],
            outsem.at[slot])

    pend_in = {}
    pend_out = {}
    for k in range(min(_NBUF, n)):
        pend_in[k] = in_copies(k)
        for d in pend_in[k]:
            d.start()
    for k in range(n):
        slot = k % _NBUF
        for d in pend_in.pop(k):
            d.wait()
        xbuf[slot] = xbuf[slot] + pebuf[slot][None, :, :]
        pend_out[k] = out_copy(k)
        pend_out[k].start()
        nk = k + _NBUF
        if nk < n:
            pend_out.pop(nk - _NBUF).wait()
            pend_in[nk] = in_copies(nk)
            for d in pend_in[nk]:
                d.start()
    for k in sorted(pend_out):
        pend_out[k].wait()


def kernel(x, pe):
    B, S, E = x.shape
    return pl.pallas_call(
        _add_pe_kernel,
        in_specs=[
            pl.BlockSpec(memory_space=pltpu.HBM),
            pl.BlockSpec(memory_space=pltpu.HBM),
        ],
        out_specs=pl.BlockSpec(memory_space=pltpu.HBM),
        out_shape=jax.ShapeDtypeStruct((B, S, E), x.dtype),
        scratch_shapes=[
            pltpu.VMEM((_NBUF, B, _C, E), x.dtype),
            pltpu.VMEM((_NBUF, _C, E), pe.dtype),
            pltpu.SemaphoreType.DMA((_NBUF,)),
            pltpu.SemaphoreType.DMA((_NBUF,)),
        ],
    )(x, pe)


# speedup vs baseline: 1.0154x; 1.0154x over previous
import jax
import jax.numpy as jnp
from jax.experimental import pallas as pl
from jax.experimental.pallas import tpu as pltpu

_C = 512
_NBUF = 3


def _add_pe_kernel(x_hbm, pe_hbm, o_hbm, xbuf, pebuf, insem, outsem):
    B, S, E = x_hbm.shape
    n = S // _C

    def in_copies(k):
        slot = k % _NBUF
        return [
            pltpu.make_async_copy(
                x_hbm.at[:, pl.ds(k * _C, _C), :], xbuf.at[slot],
                insem.at[slot]),
            pltpu.make_async_copy(
                pe_hbm.at[pl.ds(k * _C, _C), :], pebuf.at[slot],
                insem.at[slot]),
        ]

    def out_copy(k):
        slot = k % _NBUF
        return pltpu.make_async_copy(
            xbuf.at[slot], o_hbm.at[:, pl.ds(k * _C, _C), :],
            outsem.at[slot])

    pend_in = {}
    pend_out = {}
    for k in range(min(_NBUF, n)):
        pend_in[k] = in_copies(k)
        for d in pend_in[k]:
            d.start()
    for k in range(n):
        slot = k % _NBUF
        for d in pend_in.pop(k):
            d.wait()
        xbuf[slot] = xbuf[slot] + pebuf[slot][None, :, :]
        pend_out[k] = out_copy(k)
        pend_out[k].start()
        nk = k + _NBUF
        if nk < n:
            pend_out.pop(nk - _NBUF).wait()
            pend_in[nk] = in_copies(nk)
            for d in pend_in[nk]:
                d.start()
    for k in sorted(pend_out):
        pend_out[k].wait()


def kernel(x, pe):
    B, S, E = x.shape
    return pl.pallas_call(
        _add_pe_kernel,
        in_specs=[
            pl.BlockSpec(memory_space=pltpu.HBM),
            pl.BlockSpec(memory_space=pltpu.HBM),
        ],
        out_specs=pl.BlockSpec(memory_space=pltpu.HBM),
        out_shape=jax.ShapeDtypeStruct((B, S, E), x.dtype),
        scratch_shapes=[
            pltpu.VMEM((_NBUF, B, _C, E), x.dtype),
            pltpu.VMEM((_NBUF, _C, E), pe.dtype),
            pltpu.SemaphoreType.DMA((_NBUF,)),
            pltpu.SemaphoreType.DMA((_NBUF,)),
        ],
    )(x, pe)


# speedup vs baseline: 1.0199x; 1.0044x over previous
import jax
import jax.numpy as jnp
from jax.experimental import pallas as pl

_BLK = 512


def _add_pe_kernel(x_ref, pe_ref, o_ref):
    o_ref[...] = x_ref[...] + pe_ref[...][None, :, :]


def kernel(x, pe):
    B, S, E = x.shape
    blk = min(_BLK, S)
    grid = (S // blk,)
    return pl.pallas_call(
        _add_pe_kernel,
        grid=grid,
        in_specs=[
            pl.BlockSpec((B, blk, E), lambda i: (0, i, 0)),
            pl.BlockSpec((blk, E), lambda i: (i, 0)),
        ],
        out_specs=pl.BlockSpec((B, blk, E), lambda i: (0, i, 0)),
        out_shape=jax.ShapeDtypeStruct((B, S, E), x.dtype),
    )(x, pe)
